# Initial kernel scaffold; baseline (speedup 1.0000x reference)
import jax
import jax.numpy as jnp
from jax.experimental import pallas as pl


def kernel(x, W, b):
    raise NotImplementedError("write your pallas kernel here")


# speedup vs baseline: 24.8640x; 24.8640x over previous
import functools

import jax
import jax.numpy as jnp
from jax import lax
from jax.experimental import pallas as pl
from jax.experimental.pallas import tpu as pltpu
from jax.experimental.pallas import tpu_sc as plsc

KNBR = 3


def _knn_body(xrow_ref, xfull_ref, idx_ref, xt_ref):
    xr = xrow_ref[0]
    xf = xfull_ref[0]
    R = xr.shape[1]
    N = xf.shape[1]
    rowsq = jnp.sum(xr * xr, axis=0)
    colsq = jnp.sum(xf * xf, axis=0)
    cross = lax.dot_general(xr, xf, (((0,), (0,)), ((), ())),
                            preferred_element_type=jnp.float32)
    d = rowsq[:, None] + colsq[None, :] - 2.0 * cross
    base = pl.program_id(0) * N
    iota = lax.broadcasted_iota(jnp.int32, (R, N), 1)
    for k in range(KNBR):
        m = jnp.min(d, axis=1)
        cand = jnp.where(d == m[:, None], iota, jnp.int32(N))
        a = jnp.min(cand, axis=1)
        idx_ref[k, 0, 0] = (a + base)[None, :]
        d = jnp.where(iota == a[:, None], jnp.float32(jnp.inf), d)
    xt_ref[0] = xr.T


def _knn_topk(x, R):
    B, C, N = x.shape
    NB = N // R
    idx, xt = pl.pallas_call(
        _knn_body,
        grid=(B, NB),
        in_specs=[
            pl.BlockSpec((1, C, R), lambda b, j: (b, 0, j)),
            pl.BlockSpec((1, C, N), lambda b, j: (b, 0, 0)),
        ],
        out_specs=[
            pl.BlockSpec((KNBR, 1, 1, 1, R), lambda b, j: (0, b, j, 0, 0)),
            pl.BlockSpec((1, R, C), lambda b, j: (b, j, 0)),
        ],
        out_shape=[
            jax.ShapeDtypeStruct((KNBR, B, NB, 1, R), jnp.int32),
            jax.ShapeDtypeStruct((B, N, C), jnp.float32),
        ],
    )(x, x)
    return idx.reshape(KNBR, B * N), xt.reshape(B * N, C)


def _gather_stage(xt, gid):
    BN, C = xt.shape
    info = plsc.get_sparse_core_info()
    NW = info.num_cores * info.num_subcores
    CH = BN // NW
    IBLK = 128
    NBLK = CH // IBLK
    gid3 = gid.reshape(KNBR, BN // IBLK, IBLK)
    mesh = plsc.VectorSubcoreMesh(core_axis_name="c", subcore_axis_name="s")

    @functools.partial(
        pl.kernel,
        out_type=jax.ShapeDtypeStruct((KNBR, BN, C), jnp.float32),
        mesh=mesh,
        scratch_types=[
            pltpu.VMEM((NBLK, IBLK), jnp.int32),
            pltpu.VMEM((CH, C), jnp.float32),
            pltpu.SemaphoreType.DMA,
        ],
    )
    def gather_k(xt_hbm, gid_hbm, out_hbm, idx_v, rows_v, sem):
        wid = lax.axis_index("s") * info.num_cores + lax.axis_index("c")
        base = wid * CH
        blk0 = wid * NBLK
        for k in range(KNBR):
            pltpu.sync_copy(gid_hbm.at[k, pl.ds(blk0, NBLK)], idx_v)
            cps = [
                pltpu.async_copy(xt_hbm.at[idx_v.at[j]],
                                 rows_v.at[pl.ds(j * IBLK, IBLK)], sem)
                for j in range(NBLK)
            ]
            for cp in cps:
                cp.wait()
            pltpu.sync_copy(rows_v, out_hbm.at[k, pl.ds(base, CH)])

    return gather_k(xt, gid3)


def _conv_body(g_ref, w_ref, bias_ref, out_ref):
    OUT = w_ref.shape[1]
    R2 = g_ref.shape[2]
    acc = jnp.broadcast_to(bias_ref[...], (OUT, R2))
    for k in range(KNBR):
        acc = acc + lax.dot_general(w_ref[k], g_ref[k, 0],
                                    (((1,), (1,)), ((), ())),
                                    preferred_element_type=jnp.float32)
    out_ref[0] = acc


def _conv_stage(prime, w3, bias, B, N, R2):
    OUT, C = w3.shape[1], w3.shape[2]
    NB = N // R2
    return pl.pallas_call(
        _conv_body,
        grid=(B, NB),
        in_specs=[
            pl.BlockSpec((KNBR, 1, R2, C), lambda b, j: (0, b, j, 0)),
            pl.BlockSpec((KNBR, OUT, C), lambda b, j: (0, 0, 0)),
            pl.BlockSpec((OUT, 1), lambda b, j: (0, 0)),
        ],
        out_specs=pl.BlockSpec((1, OUT, R2), lambda b, j: (b, 0, j)),
        out_shape=jax.ShapeDtypeStruct((B, OUT, N), jnp.float32),
    )(prime.reshape(KNBR, B, N, C), w3, bias)


def kernel(x, W, b):
    B, C, N = x.shape
    OUT = W.shape[0]
    idx, xt = _knn_topk(x, R=256)
    prime = _gather_stage(xt, idx)
    w3 = jnp.transpose(W, (2, 0, 1))
    return _conv_stage(prime, w3, b.reshape(OUT, 1), B, N, R2=512)


# speedup vs baseline: 26.2309x; 1.0550x over previous
import functools

import jax
import jax.numpy as jnp
from jax import lax
from jax.experimental import pallas as pl
from jax.experimental.pallas import tpu as pltpu
from jax.experimental.pallas import tpu_sc as plsc

KNBR = 3


def _knn_body(xrow_ref, xfull_ref, idx_ref, xt_ref, colsq_ref):
    xr = xrow_ref[0]
    xf = xfull_ref[0]
    R = xr.shape[1]
    N = xf.shape[1]

    @pl.when(pl.program_id(1) == 0)
    def _():
        colsq_ref[...] = jnp.sum(xf * xf, axis=0, keepdims=True)

    cross = lax.dot_general(-2.0 * xr, xf, (((0,), (0,)), ((), ())),
                            preferred_element_type=jnp.float32)
    d = colsq_ref[...] + cross
    base = pl.program_id(0) * N
    iota = lax.broadcasted_iota(jnp.int32, (R, N), 1).astype(jnp.float32)
    BIG = jnp.float32(1e30)
    for k in range(KNBR):
        m = jnp.min(d, axis=1, keepdims=True)
        cand = jnp.where(d == m, iota, BIG)
        a = jnp.min(cand, axis=1, keepdims=True)
        idx_ref[k, 0, 0] = a.astype(jnp.int32) + base
        if k + 1 < KNBR:
            d = jnp.where(cand == a, BIG, d)
    xt_ref[0] = xr.T


def _knn_topk(x, R):
    B, C, N = x.shape
    NB = N // R
    idx, xt = pl.pallas_call(
        _knn_body,
        grid=(B, NB),
        in_specs=[
            pl.BlockSpec((1, C, R), lambda b, j: (b, 0, j)),
            pl.BlockSpec((1, C, N), lambda b, j: (b, 0, 0)),
        ],
        out_specs=[
            pl.BlockSpec((KNBR, 1, 1, R, 1), lambda b, j: (0, b, j, 0, 0)),
            pl.BlockSpec((1, R, C), lambda b, j: (b, j, 0)),
        ],
        out_shape=[
            jax.ShapeDtypeStruct((KNBR, B, NB, R, 1), jnp.int32),
            jax.ShapeDtypeStruct((B, N, C), jnp.float32),
        ],
        scratch_shapes=[pltpu.VMEM((1, N), jnp.float32)],
    )(x, x)
    return idx.reshape(KNBR, B * N), xt.reshape(B * N, C)


def _gather_stage(xt, gid):
    BN, C = xt.shape
    info = plsc.get_sparse_core_info()
    NW = info.num_cores * info.num_subcores
    CH = BN // NW
    IBLK = 128
    NBLK = CH // IBLK
    gid3 = gid.reshape(KNBR, BN // IBLK, IBLK)
    mesh = plsc.VectorSubcoreMesh(core_axis_name="c", subcore_axis_name="s")

    @functools.partial(
        pl.kernel,
        out_type=jax.ShapeDtypeStruct((KNBR, BN, C), jnp.float32),
        mesh=mesh,
        scratch_types=[
            pltpu.VMEM((NBLK, IBLK), jnp.int32),
            pltpu.VMEM((CH, C), jnp.float32),
            pltpu.SemaphoreType.DMA,
        ],
    )
    def gather_k(xt_hbm, gid_hbm, out_hbm, idx_v, rows_v, sem):
        wid = lax.axis_index("s") * info.num_cores + lax.axis_index("c")
        base = wid * CH
        blk0 = wid * NBLK
        for k in range(KNBR):
            pltpu.sync_copy(gid_hbm.at[k, pl.ds(blk0, NBLK)], idx_v)
            cps = [
                pltpu.async_copy(xt_hbm.at[idx_v.at[j]],
                                 rows_v.at[pl.ds(j * IBLK, IBLK)], sem)
                for j in range(NBLK)
            ]
            for cp in cps:
                cp.wait()
            pltpu.sync_copy(rows_v, out_hbm.at[k, pl.ds(base, CH)])

    return gather_k(xt, gid3)


def _conv_body(g_ref, w_ref, bias_ref, out_ref):
    OUT = w_ref.shape[1]
    R2 = g_ref.shape[2]
    acc = jnp.broadcast_to(bias_ref[...], (OUT, R2))
    for k in range(KNBR):
        acc = acc + lax.dot_general(w_ref[k], g_ref[k, 0],
                                    (((1,), (1,)), ((), ())),
                                    preferred_element_type=jnp.float32)
    out_ref[0] = acc


def _conv_stage(prime, w3, bias, B, N, R2):
    OUT, C = w3.shape[1], w3.shape[2]
    NB = N // R2
    return pl.pallas_call(
        _conv_body,
        grid=(B, NB),
        in_specs=[
            pl.BlockSpec((KNBR, 1, R2, C), lambda b, j: (0, b, j, 0)),
            pl.BlockSpec((KNBR, OUT, C), lambda b, j: (0, 0, 0)),
            pl.BlockSpec((OUT, 1), lambda b, j: (0, 0)),
        ],
        out_specs=pl.BlockSpec((1, OUT, R2), lambda b, j: (b, 0, j)),
        out_shape=jax.ShapeDtypeStruct((B, OUT, N), jnp.float32),
    )(prime.reshape(KNBR, B, N, C), w3, bias)


def kernel(x, W, b):
    B, C, N = x.shape
    OUT = W.shape[0]
    idx, xt = _knn_topk(x, R=256)
    prime = _gather_stage(xt, idx)
    w3 = jnp.transpose(W, (2, 0, 1))
    return _conv_stage(prime, w3, b.reshape(OUT, 1), B, N, R2=512)


# speedup vs baseline: 27.9995x; 1.0674x over previous
import functools

import jax
import jax.numpy as jnp
from jax import lax
from jax.experimental import pallas as pl
from jax.experimental.pallas import tpu as pltpu
from jax.experimental.pallas import tpu_sc as plsc

KNBR = 3


def _knn_body(xrow_ref, xfull_ref, idx_ref, xt_ref, colsq_ref):
    xr = xrow_ref[0]
    xf = xfull_ref[0]
    R = xr.shape[1]
    N = xf.shape[1]

    @pl.when(pl.program_id(1) == 0)
    def _():
        colsq_ref[...] = jnp.sum(xf * xf, axis=0, keepdims=True)

    cross = lax.dot_general(-2.0 * xr, xf, (((0,), (0,)), ((), ())),
                            preferred_element_type=jnp.float32)
    d = colsq_ref[...] + cross
    base = pl.program_id(0) * N
    iota = lax.broadcasted_iota(jnp.int32, (R, N), 1).astype(jnp.float32)
    BIG = jnp.float32(1e30)
    for k in range(KNBR):
        m = jnp.min(d, axis=1, keepdims=True)
        cand = jnp.where(d == m, iota, BIG)
        a = jnp.min(cand, axis=1, keepdims=True)
        idx_ref[k, 0, 0] = a.astype(jnp.int32).T + base
        if k + 1 < KNBR:
            d = jnp.where(cand == a, BIG, d)
    xt_ref[0] = xr.T


def _knn_topk(x, R):
    B, C, N = x.shape
    NB = N // R
    idx, xt = pl.pallas_call(
        _knn_body,
        grid=(B, NB),
        in_specs=[
            pl.BlockSpec((1, C, R), lambda b, j: (b, 0, j)),
            pl.BlockSpec((1, C, N), lambda b, j: (b, 0, 0)),
        ],
        out_specs=[
            pl.BlockSpec((KNBR, 1, 1, 1, R), lambda b, j: (0, b, j, 0, 0)),
            pl.BlockSpec((1, R, C), lambda b, j: (b, j, 0)),
        ],
        out_shape=[
            jax.ShapeDtypeStruct((KNBR, B, NB, 1, R), jnp.int32),
            jax.ShapeDtypeStruct((B, N, C), jnp.float32),
        ],
        scratch_shapes=[pltpu.VMEM((1, N), jnp.float32)],
    )(x, x)
    return idx.reshape(KNBR, B * N), xt.reshape(B * N, C)


def _gather_stage(xt, gid):
    BN, C = xt.shape
    info = plsc.get_sparse_core_info()
    NW = info.num_cores * info.num_subcores
    CH = BN // NW
    IBLK = 128
    NBLK = CH // IBLK
    gid3 = gid.reshape(KNBR, BN // IBLK, IBLK)
    mesh = plsc.VectorSubcoreMesh(core_axis_name="c", subcore_axis_name="s")

    @functools.partial(
        pl.kernel,
        out_type=jax.ShapeDtypeStruct((KNBR, BN, C), jnp.float32),
        mesh=mesh,
        scratch_types=[
            pltpu.VMEM((KNBR, NBLK, IBLK), jnp.int32),
            pltpu.VMEM((KNBR, CH, C), jnp.float32),
            pltpu.SemaphoreType.DMA,
            pltpu.SemaphoreType.DMA,
        ],
    )
    def gather_k(xt_hbm, gid_hbm, out_hbm, idx_v, rows_v, gsem, ssem):
        wid = lax.axis_index("s") * info.num_cores + lax.axis_index("c")
        base = wid * CH
        blk0 = wid * NBLK
        for k in range(KNBR):
            pltpu.sync_copy(gid_hbm.at[k, pl.ds(blk0, NBLK)], idx_v.at[k])
        cps = [
            pltpu.async_copy(xt_hbm.at[idx_v.at[k, j]],
                             rows_v.at[k, pl.ds(j * IBLK, IBLK)], gsem)
            for k in range(KNBR) for j in range(NBLK)
        ]
        scps = []
        for k in range(KNBR):
            for j in range(NBLK):
                cps[k * NBLK + j].wait()
            scps.append(pltpu.async_copy(rows_v.at[k],
                                         out_hbm.at[k, pl.ds(base, CH)], ssem))
        for scp in scps:
            scp.wait()

    return gather_k(xt, gid3)


def _conv_body(g_ref, w_ref, bias_ref, out_ref):
    OUT = w_ref.shape[1]
    R2 = g_ref.shape[2]
    acc = jnp.broadcast_to(bias_ref[...].T, (OUT, R2))
    for k in range(KNBR):
        acc = acc + lax.dot_general(w_ref[k], g_ref[k, 0],
                                    (((1,), (1,)), ((), ())),
                                    preferred_element_type=jnp.float32)
    out_ref[0] = acc


def _conv_stage(prime, w3, bias, B, N, R2):
    OUT, C = w3.shape[1], w3.shape[2]
    NB = N // R2
    return pl.pallas_call(
        _conv_body,
        grid=(B, NB),
        in_specs=[
            pl.BlockSpec((KNBR, 1, R2, C), lambda b, j: (0, b, j, 0)),
            pl.BlockSpec((KNBR, OUT, C), lambda b, j: (0, 0, 0)),
            pl.BlockSpec((1, OUT), lambda b, j: (0, 0)),
        ],
        out_specs=pl.BlockSpec((1, OUT, R2), lambda b, j: (b, 0, j)),
        out_shape=jax.ShapeDtypeStruct((B, OUT, N), jnp.float32),
    )(prime.reshape(KNBR, B, N, C), w3, bias)


def kernel(x, W, b):
    B, C, N = x.shape
    OUT = W.shape[0]
    idx, xt = _knn_topk(x, R=256)
    prime = _gather_stage(xt, idx)
    w3 = jnp.transpose(W, (2, 0, 1))
    return _conv_stage(prime, w3, b.reshape(1, OUT), B, N, R2=512)
